# baseline (speedup 1.0000x reference)
import functools

import jax
import jax.numpy as jnp
from jax import lax
from jax.experimental import pallas as pl
from jax.experimental.pallas import tpu as pltpu
from jax.experimental.pallas import tpu_sc as plsc
from jax.experimental.layout import Layout, with_layout_constraint

BUCKETS = 1000000
DIM = 64
LANES = 16
NUM_CORES = 2
NUM_SUBCORES = 16
NUM_WORKERS = NUM_CORES * NUM_SUBCORES
HALO = 8
W = 128


def _sc_embed(token_flat, scale_vec, table, *, b, t):
    bt = b * t
    chunk = bt // NUM_WORKERS
    n_grp = chunk // LANES
    n_win = chunk // W
    chunks_per_row = t // chunk
    mod = BUCKETS - 1

    mesh = plsc.VectorSubcoreMesh(core_axis_name="c", subcore_axis_name="s")

    @functools.partial(
        pl.kernel,
        out_type=jax.ShapeDtypeStruct((bt, DIM), jnp.float32),
        mesh=mesh,
        scratch_types=[
            pltpu.VMEM((HALO + chunk,), jnp.int32),
            pltpu.VMEM((n_win, W), jnp.int32),
            pltpu.VMEM((2, W, DIM), jnp.float32),
            pltpu.VMEM((LANES,), jnp.float32),
            pltpu.SemaphoreType.DMA,
            pltpu.SemaphoreType.DMA,
        ],
        compiler_params=pltpu.CompilerParams(use_tc_tiling_on_sc=False),
    )
    def body(tok_hbm, scale_hbm, table_hbm, out_hbm,
             tok_v, idx_v, rows_v, scale_v, gsem, osem):
        wid = lax.axis_index("s") * NUM_CORES + lax.axis_index("c")
        cpos = (wid % chunks_per_row) * chunk
        base = wid * chunk
        at_row_start = cpos == 0

        pltpu.sync_copy(scale_hbm, scale_v)

        @pl.when(at_row_start)
        def _():
            pltpu.sync_copy(tok_hbm.at[pl.ds(base, chunk)],
                            tok_v.at[pl.ds(HALO, chunk)])

        @pl.when(jnp.logical_not(at_row_start))
        def _():
            pltpu.sync_copy(tok_hbm.at[pl.ds(base - HALO, HALO + chunk)],
                            tok_v)

        lanes = lax.iota(jnp.int32, LANES)
        pos_in_row = cpos + lanes

        def hash_group(i, _):
            q = i * LANES
            t2 = tok_v[pl.ds(q + HALO, LANES)]
            t1 = tok_v[pl.ds(q + HALO - 1, LANES)]
            t0 = tok_v[pl.ds(q + HALO - 2, LANES)]
            h = 131071 * t2 + 524287 * t1 + 8191 * t0
            m = h % mod
            m = jnp.where(pos_in_row + q < 2, mod, m)
            idx_v[i // (W // LANES), pl.ds((i % (W // LANES)) * LANES,
                                           LANES)] = m
            return 0

        lax.fori_loop(0, n_grp, hash_group, 0, unroll=8)

        sv = scale_v[...]

        def fire(j, buf):
            pltpu.async_copy(table_hbm.at[idx_v.at[j]], rows_v.at[buf], gsem)

        def wait_gather(buf):
            pltpu.make_async_copy(table_hbm.at[idx_v.at[0]],
                                  rows_v.at[buf], gsem).wait()

        def scale_rows(buf):
            def one(r, _):
                for c in range(DIM // LANES):
                    rows_v[buf, r, pl.ds(c * LANES, LANES)] = (
                        rows_v[buf, r, pl.ds(c * LANES, LANES)] * sv)
                return 0
            lax.fori_loop(0, W, one, 0, unroll=4)

        def put(j, buf):
            return pltpu.async_copy(
                rows_v.at[buf],
                out_hbm.at[pl.ds(base + j * W, W)], osem)

        fire(0, 0)

        def step(jj, _):
            j0 = jj * 2
            fire(j0 + 1, 1)
            wait_gather(0)
            scale_rows(0)
            put(j0, 0).wait()

            @pl.when(j0 + 2 < n_win)
            def _():
                fire(j0 + 2, 0)

            wait_gather(1)
            scale_rows(1)
            put(j0 + 1, 1).wait()
            return 0

        lax.fori_loop(0, n_win // 2, step, 0)

    return body(token_flat, scale_vec, table)


def kernel(token_ids, embed_weight, scale):
    b, t = token_ids.shape
    table_lin = with_layout_constraint(
        embed_weight, Layout(major_to_minor=(1, 0), tiling=((8,),)))
    scale_vec = jnp.full((LANES,), scale, dtype=jnp.float32)
    tok_flat = token_ids.reshape(b * t).astype(jnp.int32)
    out = _sc_embed(tok_flat, scale_vec, table_lin, b=b, t=t)
    return out.reshape(b, t, DIM)


# speedup vs baseline: 2.0660x; 2.0660x over previous
import functools

import jax
import jax.numpy as jnp
from jax import lax
from jax.experimental import pallas as pl
from jax.experimental.pallas import tpu as pltpu
from jax.experimental.pallas import tpu_sc as plsc

BUCKETS = 1000000
DIM = 64
LANES = 16
NUM_CORES = 2
NUM_SUBCORES = 16
NUM_WORKERS = NUM_CORES * NUM_SUBCORES
HALO = 8
W = 16


def _sc_embed(token_flat, scale_vec, table3d, *, b, t):
    bt = b * t
    chunk = bt // NUM_WORKERS
    n_grp = chunk // LANES
    n_win = chunk // W
    chunks_per_row = t // chunk
    mod = BUCKETS - 1

    mesh = plsc.VectorSubcoreMesh(core_axis_name="c", subcore_axis_name="s")

    @functools.partial(
        pl.kernel,
        out_type=jax.ShapeDtypeStruct((b, DIM, t), jnp.float32),
        mesh=mesh,
        scratch_types=[
            pltpu.VMEM((HALO + chunk,), jnp.int32),
            pltpu.VMEM((n_win, W), jnp.int32),
            pltpu.VMEM((n_win, W), jnp.int32),
            pltpu.VMEM((2, W, 8, DIM), jnp.float32),
            pltpu.VMEM((DIM, chunk), jnp.float32),
            pltpu.VMEM((LANES,), jnp.float32),
            pltpu.SemaphoreType.DMA,
            pltpu.SemaphoreType.DMA,
        ],
        compiler_params=pltpu.CompilerParams(needs_layout_passes=False),
    )
    def body(tok_hbm, scale_hbm, table_hbm, out_hbm,
             tok_v, tidx_v, sidx_v, tiles_v, tout_v, scale_v, gsem, osem):
        wid = lax.axis_index("s") * NUM_CORES + lax.axis_index("c")
        seq = wid // chunks_per_row
        cpos = (wid % chunks_per_row) * chunk
        base = wid * chunk
        at_row_start = cpos == 0

        pltpu.sync_copy(scale_hbm, scale_v)

        @pl.when(at_row_start)
        def _():
            pltpu.sync_copy(tok_hbm.at[pl.ds(base, chunk)],
                            tok_v.at[pl.ds(HALO, chunk)])

        @pl.when(jnp.logical_not(at_row_start))
        def _():
            pltpu.sync_copy(tok_hbm.at[pl.ds(base - HALO, HALO + chunk)],
                            tok_v)

        lanes = lax.iota(jnp.int32, LANES)
        pos_in_row = cpos + lanes

        def hash_group(i, _):
            q = i * LANES
            t2 = tok_v[pl.ds(q + HALO, LANES)]
            t1 = tok_v[pl.ds(q + HALO - 1, LANES)]
            t0 = tok_v[pl.ds(q + HALO - 2, LANES)]
            h = 131071 * t2 + 524287 * t1 + 8191 * t0
            m = h % mod
            m = jnp.where(pos_in_row + q < 2, mod, m)
            j = i // (W // LANES)
            col = (i % (W // LANES)) * LANES
            tidx_v[j, pl.ds(col, LANES)] = m >> 3
            sidx_v[j, pl.ds(col, LANES)] = m & 7
            return 0

        lax.fori_loop(0, n_grp, hash_group, 0, unroll=8)

        sv = scale_v[...]

        def fire(j, buf):
            for g in range(W // LANES):
                tvec = tidx_v[j, pl.ds(g * LANES, LANES)]
                for l in range(LANES):
                    k = g * LANES + l
                    pltpu.async_copy(table_hbm.at[pl.ds(tvec[l], 1)],
                                     tiles_v.at[buf, pl.ds(k, 1)], gsem)

        def wait_gather(buf):
            pltpu.make_async_copy(table_hbm.at[pl.ds(0, W)],
                                  tiles_v.at[buf], gsem).wait()

        def extract(j, buf):
            for g in range(W // LANES):
                kvec = lanes + g * LANES
                svec = sidx_v[j, pl.ds(g * LANES, LANES)]
                for c in range(DIM):
                    vals = plsc.load_gather(
                        tiles_v.at[buf],
                        [kvec, svec, jnp.full((LANES,), c, jnp.int32)])
                    tout_v[c, pl.ds(j * W + g * LANES, LANES)] = vals * sv

        fire(0, 0)

        def step(jj, _):
            j0 = jj * 2
            fire(j0 + 1, 1)
            wait_gather(0)
            extract(j0, 0)

            @pl.when(j0 + 2 < n_win)
            def _():
                fire(j0 + 2, 0)

            wait_gather(1)
            extract(j0 + 1, 1)
            return 0

        lax.fori_loop(0, n_win // 2, step, 0)

        pltpu.async_copy(tout_v,
                         out_hbm.at[seq, slice(None), pl.ds(cpos, chunk)],
                         osem).wait()

    return body(token_flat, scale_vec, table3d)


def kernel(token_ids, embed_weight, scale):
    b, t = token_ids.shape
    table3d = embed_weight.reshape(BUCKETS // 8, 8, DIM)
    scale_vec = jnp.full((LANES,), scale, dtype=jnp.float32)
    tok_flat = token_ids.reshape(b * t).astype(jnp.int32)
    out_t = _sc_embed(tok_flat, scale_vec, table3d, b=b, t=t)
    return out_t.transpose(0, 2, 1)


# speedup vs baseline: 2.1724x; 1.0515x over previous
import functools

import jax
import jax.numpy as jnp
from jax import lax
from jax.experimental import pallas as pl
from jax.experimental.pallas import tpu as pltpu
from jax.experimental.pallas import tpu_sc as plsc

BUCKETS = 1000000
DIM = 64
LANES = 16
NUM_CORES = 2
NUM_SUBCORES = 16
NUM_WORKERS = NUM_CORES * NUM_SUBCORES
HALO = 8
W = 16
NBUF = 4


def _sc_embed(token_flat, scale_vec, table3d, *, b, t):
    bt = b * t
    chunk = bt // NUM_WORKERS
    n_grp = chunk // LANES
    n_win = chunk // W
    chunks_per_row = t // chunk
    mod = BUCKETS - 1

    mesh = plsc.VectorSubcoreMesh(core_axis_name="c", subcore_axis_name="s")

    @functools.partial(
        pl.kernel,
        out_type=jax.ShapeDtypeStruct((b, DIM, t), jnp.float32),
        mesh=mesh,
        scratch_types=[
            pltpu.VMEM((HALO + chunk,), jnp.int32),
            pltpu.VMEM((n_win, W), jnp.int32),
            pltpu.VMEM((n_win, W), jnp.int32),
            pltpu.VMEM((NBUF, W, 8, DIM), jnp.float32),
            pltpu.VMEM((DIM, chunk // 2), jnp.float32),
            pltpu.VMEM((LANES,), jnp.float32),
            pltpu.SemaphoreType.DMA,
            pltpu.SemaphoreType.DMA,
        ],
        compiler_params=pltpu.CompilerParams(needs_layout_passes=False),
    )
    def body(tok_hbm, scale_hbm, table_hbm, out_hbm,
             tok_v, tidx_v, sidx_v, tiles_v, tout_v, scale_v, gsem, osem):
        wid = lax.axis_index("s") * NUM_CORES + lax.axis_index("c")
        seq = wid // chunks_per_row
        cpos = (wid % chunks_per_row) * chunk
        base = wid * chunk
        at_row_start = cpos == 0

        pltpu.sync_copy(scale_hbm, scale_v)

        @pl.when(at_row_start)
        def _():
            pltpu.sync_copy(tok_hbm.at[pl.ds(base, chunk)],
                            tok_v.at[pl.ds(HALO, chunk)])

        @pl.when(jnp.logical_not(at_row_start))
        def _():
            pltpu.sync_copy(tok_hbm.at[pl.ds(base - HALO, HALO + chunk)],
                            tok_v)

        lanes = lax.iota(jnp.int32, LANES)
        pos_in_row = cpos + lanes

        def hash_group(i, _):
            q = i * LANES
            t2 = tok_v[pl.ds(q + HALO, LANES)]
            t1 = tok_v[pl.ds(q + HALO - 1, LANES)]
            t0 = tok_v[pl.ds(q + HALO - 2, LANES)]
            h = 131071 * t2 + 524287 * t1 + 8191 * t0
            m = h % mod
            m = jnp.where(pos_in_row + q < 2, mod, m)
            j = i // (W // LANES)
            col = (i % (W // LANES)) * LANES
            tidx_v[j, pl.ds(col, LANES)] = m >> 3
            sidx_v[j, pl.ds(col, LANES)] = m & 7
            return 0

        lax.fori_loop(0, n_grp, hash_group, 0, unroll=8)

        sv = scale_v[...]

        def fire(j, buf):
            tvec = tidx_v[j, pl.ds(0, LANES)]
            for l in range(LANES):
                pltpu.async_copy(table_hbm.at[pl.ds(tvec[l], 1)],
                                 tiles_v.at[buf, pl.ds(l, 1)], gsem)

        def wait_gather():
            pltpu.make_async_copy(table_hbm.at[pl.ds(0, W)],
                                  tiles_v.at[0], gsem).wait()

        def extract(j, buf):
            svec = sidx_v[j, pl.ds(0, LANES)]
            bvec = jnp.full((LANES,), buf, jnp.int32)
            off = lax.rem(j, n_win // 2) * LANES
            for c in range(DIM):
                vals = plsc.load_gather(
                    tiles_v,
                    [bvec, lanes, svec, jnp.full((LANES,), c, jnp.int32)])
                tout_v[c, pl.ds(off, LANES)] = vals * sv

        def put(half):
            pltpu.async_copy(
                tout_v,
                out_hbm.at[seq, slice(None),
                           pl.ds(cpos + half * (chunk // 2), chunk // 2)],
                osem).wait()

        for p in range(NBUF - 1):
            fire(p, p)

        def step(j, buf):
            wait_gather()
            extract(j, buf)

            @pl.when(j + NBUF - 1 < n_win)
            def _():
                fire(j + NBUF - 1, lax.rem(buf + NBUF - 1, NBUF))

            @pl.when(j == n_win // 2 - 1)
            def _():
                put(0)

            buf = buf + 1
            return jnp.where(buf == NBUF, 0, buf)

        lax.fori_loop(0, n_win, step, jnp.int32(0))
        put(1)

    return body(token_flat, scale_vec, table3d)


def kernel(token_ids, embed_weight, scale):
    b, t = token_ids.shape
    table3d = embed_weight.reshape(BUCKETS // 8, 8, DIM)
    scale_vec = jnp.full((LANES,), scale, dtype=jnp.float32)
    tok_flat = token_ids.reshape(b * t).astype(jnp.int32)
    out_t = _sc_embed(tok_flat, scale_vec, table3d, b=b, t=t)
    return out_t.transpose(0, 2, 1)
